# baseline (speedup 1.0000x reference)
import functools

import jax
import jax.numpy as jnp
from jax import lax
from jax.experimental import pallas as pl
from jax.experimental.pallas import tpu as pltpu
from jax.experimental.pallas import tpu_sc as plsc

_B, _L, _D = 4, 4096, 1024
_N = _B * _L

_NC, _NS = 2, 16
_NW = _NC * _NS
_ROWS_PER_W = _N // _NW
_CHUNK = 16
_NCHUNKS = _ROWS_PER_W // _CHUNK
_NB = 6
_DEPTH = 3

_mesh = plsc.VectorSubcoreMesh(core_axis_name="c", subcore_axis_name="s")


@functools.partial(
    pl.kernel,
    mesh=_mesh,
    out_type=jax.ShapeDtypeStruct((_N, _D), jnp.float32),
    scratch_types=[
        pltpu.VMEM((_NCHUNKS, _CHUNK), jnp.int32),
        pltpu.VMEM((_NB, _CHUNK, _D), jnp.float32),
        pltpu.SemaphoreType.DMA,
        pltpu.SemaphoreType.DMA,
        pltpu.SemaphoreType.DMA,
        pltpu.SemaphoreType.DMA,
        pltpu.SemaphoreType.DMA,
        pltpu.SemaphoreType.DMA,
        pltpu.SemaphoreType.DMA,
        pltpu.SemaphoreType.DMA,
        pltpu.SemaphoreType.DMA,
        pltpu.SemaphoreType.DMA,
        pltpu.SemaphoreType.DMA,
        pltpu.SemaphoreType.DMA,
    ],
)
def _interleave(x_hbm, perm_hbm, out_hbm, idx_v, rows_v,
                g0, g1, g2, g3, g4, g5, w0, w1, w2, w3, w4, w5):
    wid = lax.axis_index("s") * _NC + lax.axis_index("c")
    base = wid * _ROWS_PER_W
    pltpu.sync_copy(perm_hbm.at[wid], idx_v)
    gsem = (g0, g1, g2, g3, g4, g5)
    wsem = (w0, w1, w2, w3, w4, w5)

    def gather(c):
        b = c % _NB
        return pltpu.async_copy(
            x_hbm.at[idx_v.at[c]], rows_v.at[b], gsem[b])

    def write(c):
        b = c % _NB
        return pltpu.async_copy(
            rows_v.at[b], out_hbm.at[pl.ds(base + c * _CHUNK, _CHUNK)], wsem[b])

    gathers = [None] * _NCHUNKS
    writes = [None] * _NCHUNKS
    for c in range(min(_DEPTH, _NCHUNKS)):
        gathers[c] = gather(c)
    for c in range(_NCHUNKS):
        gathers[c].wait()
        writes[c] = write(c)
        n = c + _DEPTH
        if n < _NCHUNKS:
            if n - _NB >= 0:
                writes[n - _NB].wait()
            gathers[n] = gather(n)
    for c in range(max(0, _NCHUNKS - _NB), _NCHUNKS):
        writes[c].wait()


def kernel(x, perm):
    xf = x.reshape(_N, _D)
    out = _interleave(xf, perm.reshape(_NW, _NCHUNKS, _CHUNK))
    return out.reshape(_B, _L, _D)
